# Initial kernel scaffold; baseline (speedup 1.0000x reference)
import jax
import jax.numpy as jnp
from jax.experimental import pallas as pl


def kernel(x, edge_index, batch, y, W1, b1, W2, b2, W3, b3, W4, b4):
    raise NotImplementedError("write your pallas kernel here")


# speedup vs baseline: 114.3641x; 114.3641x over previous
import functools

import jax
import jax.numpy as jnp
from jax import lax
from jax.experimental import pallas as pl
from jax.experimental.pallas import tpu as pltpu
from jax.experimental.pallas import tpu_sc as plsc

N = 50000
E = 800000
G = 128

NC = 2
NS = 16
NW = NC * NS

NPAD = 50176
SL = NPAD // NS
EPAD = 802816
EPT = EPAD // NW
NCH = EPT // 128

ROWS = 392
TCG = 49

_mesh = plsc.VectorSubcoreMesh(
    core_axis_name="c", subcore_axis_name="s", num_cores=NC, num_subcores=NS)


def _zero_fill(buf, n16):
    def body(i, _):
        buf[pl.ds(i * 16, 16)] = jnp.zeros((16,), jnp.float32)
        return 0
    lax.fori_loop(0, n16, body, 0)


def _scatter_chunks(vals_at, acc_sh, didx_v, sem):
    def fire(j, _):
        pltpu.async_copy(vals_at(j), acc_sh.at[didx_v.at[j]], sem, add=True)
        return 0
    lax.fori_loop(0, NCH, fire, 0)

    def drain(j, _):
        pltpu.make_async_copy(vals_at(0), acc_sh.at[didx_v.at[0]], sem).wait()
        return 0
    lax.fori_loop(0, NCH, drain, 0)


@functools.partial(
    pl.kernel,
    out_type=jax.ShapeDtypeStruct((2 * NPAD,), jnp.float32),
    mesh=_mesh,
    scratch_types=[
        pltpu.VMEM((NCH, 128), jnp.int32),
        pltpu.VMEM((128,), jnp.float32),
        pltpu.VMEM((SL,), jnp.float32),
        pltpu.VMEM_SHARED((NPAD,), jnp.float32),
        pltpu.SemaphoreType.DMA,
    ],
)
def _deg_kernel(dst_hbm, out_hbm, didx_v, ones_v, zbuf_v, acc_sh, sem):
    c = lax.axis_index("c")
    s = lax.axis_index("s")
    wid = c * NS + s

    _zero_fill(zbuf_v, SL // 16)
    pltpu.sync_copy(zbuf_v, acc_sh.at[pl.ds(s * SL, SL)])

    def ofill(i, _):
        ones_v[pl.ds(i * 16, 16)] = jnp.ones((16,), jnp.float32)
        return 0
    lax.fori_loop(0, 8, ofill, 0)
    plsc.subcore_barrier()

    pltpu.sync_copy(dst_hbm.at[wid], didx_v)
    _scatter_chunks(lambda j: ones_v, acc_sh, didx_v, sem)
    plsc.subcore_barrier()

    pltpu.sync_copy(acc_sh.at[pl.ds(s * SL, SL)], zbuf_v)
    pltpu.sync_copy(zbuf_v, out_hbm.at[pl.ds(c * NPAD + s * SL, SL)])


@functools.partial(
    pl.kernel,
    out_type=jax.ShapeDtypeStruct((2 * NPAD,), jnp.float32),
    mesh=_mesh,
    scratch_types=[
        pltpu.VMEM((EPT,), jnp.int32),
        pltpu.VMEM((NCH, 128), jnp.int32),
        pltpu.VMEM((EPT,), jnp.float32),
        pltpu.VMEM((ROWS, 128), jnp.float32),
        pltpu.VMEM_SHARED((NPAD,), jnp.float32),
        pltpu.SemaphoreType.DMA,
    ],
    compiler_params=pltpu.CompilerParams(needs_layout_passes=False),
)
def _gs_kernel(src_hbm, dst_hbm, a_hbm, out_hbm,
               sidx_v, didx_v, gvals_v, a_v, acc_sh, sem):
    c = lax.axis_index("c")
    s = lax.axis_index("s")
    wid = c * NS + s

    _zero_fill(gvals_v, SL // 16)
    pltpu.sync_copy(gvals_v.at[pl.ds(0, SL)], acc_sh.at[pl.ds(s * SL, SL)])
    plsc.subcore_barrier()

    pltpu.sync_copy(a_hbm, a_v)
    pltpu.sync_copy(src_hbm.at[pl.ds(wid * EPT, EPT)], sidx_v)
    pltpu.sync_copy(dst_hbm.at[wid], didx_v)

    def gather(k, _):
        idx16 = sidx_v[pl.ds(k * 16, 16)]
        gvals_v[pl.ds(k * 16, 16)] = plsc.load_gather(
            a_v, [lax.shift_right_logical(idx16, 7),
                  lax.bitwise_and(idx16, 127)])
        return 0
    lax.fori_loop(0, EPT // 16, gather, 0)

    _scatter_chunks(lambda j: gvals_v.at[pl.ds(j * 128, 128)],
                    acc_sh, didx_v, sem)
    plsc.subcore_barrier()

    pltpu.sync_copy(acc_sh.at[pl.ds(s * SL, SL)], gvals_v.at[pl.ds(0, SL)])
    pltpu.sync_copy(gvals_v.at[pl.ds(0, SL)],
                    out_hbm.at[pl.ds(c * NPAD + s * SL, SL)])


def _prep_body(h0_ref, h1_ref, x_ref, a_ref, deg_ref):
    d = h0_ref[...] + h1_ref[...] + 1.0
    deg_ref[...] = d
    a_ref[...] = x_ref[...] * lax.rsqrt(d)


def _prep_call(h0, h1, x2d):
    blk = pl.BlockSpec((8, 128), lambda i: (i, 0))
    return pl.pallas_call(
        _prep_body,
        grid=(TCG,),
        in_specs=[blk, blk, blk],
        out_specs=[blk, blk],
        out_shape=[
            jax.ShapeDtypeStruct((ROWS, 128), jnp.float32),
            jax.ShapeDtypeStruct((ROWS, 128), jnp.float32),
        ],
    )(h0, h1, x2d)


def _head_body(t0_ref, t1_ref, deg_ref, x_ref, b_ref, w1t_ref, b1t_ref,
               w2_ref, b2_ref, w3a_ref, w3b_ref, b3_ref, w4_ref, b4_ref,
               y_ref, out_ref, acc, cnt):
    i = pl.program_id(0)

    @pl.when(i == 0)
    def _():
        acc[...] = jnp.zeros_like(acc)
        cnt[...] = jnp.zeros_like(cnt)

    d = deg_ref[...]
    t = t0_ref[...] + t1_ref[...]
    s = lax.rsqrt(d) * t + x_ref[...] / d
    b = b_ref[...]

    s_full = jnp.concatenate(
        [jnp.broadcast_to(s[r:r + 1, :], (G, 128)) for r in range(8)],
        axis=1)
    b_full = jnp.concatenate(
        [jnp.broadcast_to(b[r:r + 1, :], (G, 128)) for r in range(8)],
        axis=1)

    mat_t = jnp.maximum(
        jnp.broadcast_to(w1t_ref[...], (G, 1024)) * s_full
        + jnp.broadcast_to(b1t_ref[...], (G, 1024)), 0.0)

    gid = lax.broadcasted_iota(jnp.int32, (G, 1024), 0)
    oh = (gid == b_full).astype(jnp.float32)

    acc[...] += lax.dot_general(
        oh, mat_t, (((1,), (1,)), ((), ())),
        precision=lax.Precision.HIGHEST,
        preferred_element_type=jnp.float32)
    cnt[...] += jnp.sum(oh, axis=1, keepdims=True)

    @pl.when(i == TCG - 1)
    def _():
        pooled = acc[...] / jnp.maximum(cnt[...], 1.0)
        emb = jnp.maximum(
            jnp.dot(pooled, w2_ref[...],
                    precision=lax.Precision.HIGHEST) + b2_ref[...], 0.0)
        h3 = jnp.maximum(
            jnp.dot(emb, w3a_ref[...], precision=lax.Precision.HIGHEST)
            + y_ref[...] * w3b_ref[...] + b3_ref[...], 0.0)
        out_ref[...] = (
            jnp.dot(h3, w4_ref[...], precision=lax.Precision.HIGHEST)
            + b4_ref[...])


def _head_call(t0, t1, deg2d, x2d, b2d, w1t, b1t, w2, b2r, w3a, w3b, b3r,
               w4, b4r, ycol):
    blk = pl.BlockSpec((8, 128), lambda i: (i, 0))
    full = lambda shape: pl.BlockSpec(shape, lambda i: tuple(0 for _ in shape))
    return pl.pallas_call(
        _head_body,
        grid=(TCG,),
        in_specs=[
            blk, blk, blk, blk, blk,
            full((G, 1)), full((G, 1)),
            full((128, 64)), full((1, 64)),
            full((64, 32)), full((1, 32)), full((1, 32)),
            full((32, 1)), full((1, 1)),
            full((G, 1)),
        ],
        out_specs=pl.BlockSpec((G, 1), lambda i: (0, 0)),
        out_shape=jax.ShapeDtypeStruct((G, 1), jnp.float32),
        scratch_shapes=[
            pltpu.VMEM((G, G), jnp.float32),
            pltpu.VMEM((G, 1), jnp.float32),
        ],
    )(t0, t1, deg2d, x2d, b2d, w1t, b1t, w2, b2r, w3a, w3b, b3r, w4, b4r,
      ycol)


def kernel(x, edge_index, batch, y, W1, b1, W2, b2, W3, b3, W4, b4):
    pad_idx = jnp.full((EPAD - E,), NPAD - 1, dtype=jnp.int32)
    src_p = jnp.concatenate([edge_index[0], pad_idx])
    dst3d = jnp.concatenate([edge_index[1], pad_idx]).reshape(NW, NCH, 128)

    x_flat = jnp.pad(x.reshape(-1), (0, NPAD - N))
    batch_p = jnp.pad(batch, (0, NPAD - N), constant_values=G)

    hist = _deg_kernel(dst3d)
    a2d, deg2d = _prep_call(hist[:NPAD].reshape(ROWS, 128),
                            hist[NPAD:].reshape(ROWS, 128),
                            x_flat.reshape(ROWS, 128))
    t = _gs_kernel(src_p, dst3d, a2d)

    out = _head_call(
        t[:NPAD].reshape(ROWS, 128),
        t[NPAD:].reshape(ROWS, 128),
        deg2d,
        x_flat.reshape(ROWS, 128),
        batch_p.reshape(ROWS, 128),
        W1.reshape(G, 1), b1.reshape(G, 1),
        W2, b2.reshape(1, 64),
        W3[:64], W3[64:65], b3.reshape(1, 32),
        W4, b4.reshape(1, 1),
        y.reshape(G, 1),
    )
    return out.reshape(-1)


# speedup vs baseline: 139.6436x; 1.2210x over previous
import functools

import jax
import jax.numpy as jnp
from jax import lax
from jax.experimental import pallas as pl
from jax.experimental.pallas import tpu as pltpu
from jax.experimental.pallas import tpu_sc as plsc

N = 50000
E = 800000
G = 128

NC = 2
NS = 16
NW = NC * NS

NPAD = 50176
ROWS = NPAD // 128
SL = NPAD // NS
EPAD = 802816
EPT = EPAD // NW
NCH = EPT // 128
TCG = ROWS // 8

_mesh = plsc.VectorSubcoreMesh(
    core_axis_name="c", subcore_axis_name="s", num_cores=NC, num_subcores=NS)
_sc_params = pltpu.CompilerParams(needs_layout_passes=False)


def _fill(buf, n16, value):
    def body(i, _):
        buf[pl.ds(i * 16, 16)] = jnp.full((16,), value, jnp.float32)
        return 0
    lax.fori_loop(0, n16, body, 0)


def _fire_drain(vals_at, acc_sh, didx_v, sem, nch):
    def fire(j, _):
        pltpu.async_copy(vals_at(j), acc_sh.at[didx_v.at[j]], sem, add=True)
        return 0
    lax.fori_loop(0, nch, fire, 0)

    def drain(j, _):
        pltpu.make_async_copy(vals_at(0), acc_sh.at[didx_v.at[0]], sem).wait()
        return 0
    lax.fori_loop(0, nch, drain, 0)


@functools.partial(
    pl.kernel,
    out_type=(jax.ShapeDtypeStruct((2 * NPAD,), jnp.float32),
              jax.ShapeDtypeStruct((256,), jnp.float32)),
    mesh=_mesh,
    scratch_types=[
        pltpu.VMEM((NCH, 128), jnp.int32),
        pltpu.VMEM((128,), jnp.float32),
        pltpu.VMEM((SL,), jnp.float32),
        pltpu.VMEM_SHARED((NPAD,), jnp.float32),
        pltpu.VMEM_SHARED((256,), jnp.float32),
        pltpu.SemaphoreType.DMA,
    ],
    compiler_params=_sc_params,
)
def _deg_kernel(dst3d_hbm, batch2d_hbm, hist_hbm, cnt_hbm,
                didx_v, ones_v, zbuf_v, deg_sh, cnt_sh, sem):
    c = lax.axis_index("c")
    s = lax.axis_index("s")
    wid = c * NS + s

    _fill(zbuf_v, SL // 16, 0.0)
    pltpu.sync_copy(zbuf_v, deg_sh.at[pl.ds(s * SL, SL)])

    @pl.when(jnp.logical_and(c == 0, s == 0))
    def _():
        pltpu.sync_copy(zbuf_v.at[pl.ds(0, 256)], cnt_sh)

    _fill(ones_v, 8, 1.0)
    plsc.subcore_barrier()

    pltpu.sync_copy(dst3d_hbm.at[wid], didx_v)
    _fire_drain(lambda j: ones_v, deg_sh, didx_v, sem, NCH)

    nrows = jnp.where(s < NS - 1, 24, 32)

    @pl.when(jnp.logical_and(c == 0, s < NS - 1))
    def _():
        pltpu.sync_copy(batch2d_hbm.at[pl.ds(s * 24, 24)],
                        didx_v.at[pl.ds(0, 24)])

    @pl.when(jnp.logical_and(c == 0, s == NS - 1))
    def _():
        pltpu.sync_copy(batch2d_hbm.at[pl.ds((NS - 1) * 24, 32)],
                        didx_v.at[pl.ds(0, 32)])

    @pl.when(c == 0)
    def _():
        _fire_drain(lambda j: ones_v, cnt_sh, didx_v, sem, nrows)

    plsc.subcore_barrier()

    pltpu.sync_copy(deg_sh.at[pl.ds(s * SL, SL)], zbuf_v)
    pltpu.sync_copy(zbuf_v, hist_hbm.at[pl.ds(c * NPAD + s * SL, SL)])

    @pl.when(jnp.logical_and(c == 0, s == 0))
    def _():
        pltpu.sync_copy(cnt_sh, zbuf_v.at[pl.ds(0, 256)])
        pltpu.sync_copy(zbuf_v.at[pl.ds(0, 256)], cnt_hbm)


@functools.partial(
    pl.kernel,
    out_type=jax.ShapeDtypeStruct((2 * NPAD,), jnp.float32),
    mesh=_mesh,
    scratch_types=[
        pltpu.VMEM((NCH, 128), jnp.int32),
        pltpu.VMEM((NCH, 128), jnp.int32),
        pltpu.VMEM((EPT,), jnp.float32),
        pltpu.VMEM_SHARED((NPAD,), jnp.float32),
        pltpu.VMEM_SHARED((NPAD,), jnp.float32),
        pltpu.SemaphoreType.DMA,
    ],
    compiler_params=_sc_params,
)
def _gs_kernel(src3d_hbm, dst3d_hbm, hist_hbm, x_hbm, t_hbm,
               sidx_v, didx_v, gvals_v, a_sh, t_sh, sem):
    c = lax.axis_index("c")
    s = lax.axis_index("s")
    wid = c * NS + s

    _fill(gvals_v, SL // 16, 0.0)
    pltpu.sync_copy(gvals_v.at[pl.ds(0, SL)], t_sh.at[pl.ds(s * SL, SL)])

    pltpu.sync_copy(hist_hbm.at[pl.ds(s * SL, SL)], gvals_v.at[pl.ds(0, SL)])
    pltpu.sync_copy(hist_hbm.at[pl.ds(NPAD + s * SL, SL)],
                    gvals_v.at[pl.ds(SL, SL)])
    pltpu.sync_copy(x_hbm.at[pl.ds(s * SL, SL)], gvals_v.at[pl.ds(2 * SL, SL)])

    def newton(k, _):
        d = (gvals_v[pl.ds(k * 16, 16)]
             + gvals_v[pl.ds(SL + k * 16, 16)] + 1.0)
        i = jnp.int32(0x5F3759DF) - lax.shift_right_logical(
            plsc.bitcast(d, jnp.int32), 1)
        y = plsc.bitcast(i, jnp.float32)
        y = y * (1.5 - 0.5 * d * y * y)
        y = y * (1.5 - 0.5 * d * y * y)
        y = y * (1.5 - 0.5 * d * y * y)
        gvals_v[pl.ds(3 * SL + k * 16, 16)] = (
            gvals_v[pl.ds(2 * SL + k * 16, 16)] * y)
        return 0
    lax.fori_loop(0, SL // 16, newton, 0)

    pltpu.sync_copy(gvals_v.at[pl.ds(3 * SL, SL)], a_sh.at[pl.ds(s * SL, SL)])
    plsc.subcore_barrier()

    pltpu.sync_copy(src3d_hbm.at[wid], sidx_v)
    pltpu.sync_copy(dst3d_hbm.at[wid], didx_v)

    def gfire(j, _):
        pltpu.async_copy(a_sh.at[sidx_v.at[j]],
                         gvals_v.at[pl.ds(j * 128, 128)], sem)
        return 0
    lax.fori_loop(0, NCH, gfire, 0)

    def gdrain(j, _):
        pltpu.make_async_copy(a_sh.at[sidx_v.at[0]],
                              gvals_v.at[pl.ds(0, 128)], sem).wait()
        return 0
    lax.fori_loop(0, NCH, gdrain, 0)

    _fire_drain(lambda j: gvals_v.at[pl.ds(j * 128, 128)],
                t_sh, didx_v, sem, NCH)
    plsc.subcore_barrier()

    pltpu.sync_copy(t_sh.at[pl.ds(s * SL, SL)], gvals_v.at[pl.ds(0, SL)])
    pltpu.sync_copy(gvals_v.at[pl.ds(0, SL)],
                    t_hbm.at[pl.ds(c * NPAD + s * SL, SL)])


def _head_body(t0_ref, t1_ref, h0_ref, h1_ref, x_ref, b_ref, cnt_ref,
               w1t_ref, b1t_ref, w2_ref, b2_ref, w3a_ref, w3b_ref, b3_ref,
               w4_ref, b4_ref, y_ref, out_ref, acc):
    i = pl.program_id(0)

    @pl.when(i == 0)
    def _():
        acc[...] = jnp.zeros_like(acc)

    d = h0_ref[...] + h1_ref[...] + 1.0
    t = t0_ref[...] + t1_ref[...]
    s = lax.rsqrt(d) * t + x_ref[...] / d
    b = b_ref[...].astype(jnp.bfloat16)

    s_full = jnp.concatenate(
        [jnp.broadcast_to(s[r:r + 1, :], (G, 128)) for r in range(8)],
        axis=1)
    b_full = jnp.concatenate(
        [jnp.broadcast_to(b[r:r + 1, :], (G, 128)) for r in range(8)],
        axis=1)

    mat_t = jnp.maximum(
        jnp.broadcast_to(w1t_ref[...], (G, 1024)) * s_full
        + jnp.broadcast_to(b1t_ref[...], (G, 1024)),
        0.0).astype(jnp.bfloat16)

    gid = lax.broadcasted_iota(jnp.int32, (G, 1), 0).astype(jnp.bfloat16)
    gid = jnp.broadcast_to(gid, (G, 1024))
    oh = (gid == b_full).astype(jnp.bfloat16)

    acc[...] += lax.dot_general(
        oh, mat_t, (((1,), (1,)), ((), ())),
        preferred_element_type=jnp.float32)

    @pl.when(i == TCG - 1)
    def _():
        pooled = acc[...] / jnp.maximum(cnt_ref[...], 1.0)
        emb = jnp.maximum(
            jnp.dot(pooled, w2_ref[...],
                    precision=lax.Precision.HIGHEST) + b2_ref[...], 0.0)
        h3 = jnp.maximum(
            jnp.dot(emb, w3a_ref[...], precision=lax.Precision.HIGHEST)
            + y_ref[...] * w3b_ref[...] + b3_ref[...], 0.0)
        out_ref[...] = (
            jnp.dot(h3, w4_ref[...], precision=lax.Precision.HIGHEST)
            + b4_ref[...])


def _head_call(t0, t1, h0, h1, x2d, b2d, cnt_col, w1t, b1t, w2, b2r,
               w3a, w3b, b3r, w4, b4r, ycol):
    blk = pl.BlockSpec((8, 128), lambda i: (i, 0))
    full = lambda shape: pl.BlockSpec(shape, lambda i: tuple(0 for _ in shape))
    return pl.pallas_call(
        _head_body,
        grid=(TCG,),
        in_specs=[
            blk, blk, blk, blk, blk, blk,
            full((G, 1)),
            full((G, 1)), full((G, 1)),
            full((128, 64)), full((1, 64)),
            full((64, 32)), full((1, 32)), full((1, 32)),
            full((32, 1)), full((1, 1)),
            full((G, 1)),
        ],
        out_specs=pl.BlockSpec((G, 1), lambda i: (0, 0)),
        out_shape=jax.ShapeDtypeStruct((G, 1), jnp.float32),
        scratch_shapes=[pltpu.VMEM((G, G), jnp.float32)],
    )(t0, t1, h0, h1, x2d, b2d, cnt_col, w1t, b1t, w2, b2r, w3a, w3b, b3r,
      w4, b4r, ycol)


def kernel(x, edge_index, batch, y, W1, b1, W2, b2, W3, b3, W4, b4):
    pad_idx = jnp.full((EPAD - E,), NPAD - 1, dtype=jnp.int32)
    src3d = jnp.concatenate([edge_index[0], pad_idx]).reshape(NW, NCH, 128)
    dst3d = jnp.concatenate([edge_index[1], pad_idx]).reshape(NW, NCH, 128)

    x_flat = jnp.pad(x.reshape(-1), (0, NPAD - N))
    batch_p = jnp.pad(batch, (0, NPAD - N), constant_values=G)

    hist, cnt = _deg_kernel(dst3d, batch_p.reshape(ROWS, 128))
    t = _gs_kernel(src3d, dst3d, hist, x_flat)

    out = _head_call(
        t[:NPAD].reshape(ROWS, 128),
        t[NPAD:].reshape(ROWS, 128),
        hist[:NPAD].reshape(ROWS, 128),
        hist[NPAD:].reshape(ROWS, 128),
        x_flat.reshape(ROWS, 128),
        batch_p.reshape(ROWS, 128),
        cnt[:G].reshape(G, 1),
        W1.reshape(G, 1), b1.reshape(G, 1),
        W2, b2.reshape(1, 64),
        W3[:64], W3[64:65], b3.reshape(1, 32),
        W4, b4.reshape(1, 1),
        y.reshape(G, 1),
    )
    return out.reshape(-1)


# speedup vs baseline: 139.9522x; 1.0022x over previous
import functools

import jax
import jax.numpy as jnp
from jax import lax
from jax.experimental import pallas as pl
from jax.experimental.pallas import tpu as pltpu
from jax.experimental.pallas import tpu_sc as plsc

N = 50000
E = 800000
G = 128

NC = 2
NS = 16
NW = NC * NS

NPAD = 50176
ROWS = NPAD // 128
SL = NPAD // NS
EPAD = 802816
EPT = EPAD // NW
NCH = EPT // 128
TCG = ROWS // 8

_mesh = plsc.VectorSubcoreMesh(
    core_axis_name="c", subcore_axis_name="s", num_cores=NC, num_subcores=NS)
_sc_params = pltpu.CompilerParams(needs_layout_passes=False)


def _fill(buf, n16, value):
    def body(i, _):
        buf[pl.ds(i * 16, 16)] = jnp.full((16,), value, jnp.float32)
        return 0
    lax.fori_loop(0, n16, body, 0)


def _fire_drain(vals_at, acc_sh, didx_v, sem, nch):
    def fire(j, _):
        pltpu.async_copy(vals_at(j), acc_sh.at[didx_v.at[j]], sem, add=True)
        return 0
    lax.fori_loop(0, nch, fire, 0)

    def drain(j, _):
        pltpu.make_async_copy(vals_at(0), acc_sh.at[didx_v.at[0]], sem).wait()
        return 0
    lax.fori_loop(0, nch, drain, 0)


@functools.partial(
    pl.kernel,
    out_type=(jax.ShapeDtypeStruct((2 * NPAD,), jnp.float32),
              jax.ShapeDtypeStruct((256,), jnp.float32)),
    mesh=_mesh,
    scratch_types=[
        pltpu.VMEM((NCH, 128), jnp.int32),
        pltpu.VMEM((128,), jnp.float32),
        pltpu.VMEM((SL,), jnp.float32),
        pltpu.VMEM_SHARED((NPAD,), jnp.float32),
        pltpu.VMEM_SHARED((256,), jnp.float32),
        pltpu.SemaphoreType.DMA,
    ],
    compiler_params=_sc_params,
)
def _deg_kernel(dst3d_hbm, batch2d_hbm, hist_hbm, cnt_hbm,
                didx_v, ones_v, zbuf_v, deg_sh, cnt_sh, sem):
    c = lax.axis_index("c")
    s = lax.axis_index("s")
    wid = c * NS + s

    _fill(zbuf_v, SL // 16, 0.0)
    pltpu.sync_copy(zbuf_v, deg_sh.at[pl.ds(s * SL, SL)])

    @pl.when(jnp.logical_and(c == 0, s == 0))
    def _():
        pltpu.sync_copy(zbuf_v.at[pl.ds(0, 256)], cnt_sh)

    _fill(ones_v, 8, 1.0)
    plsc.subcore_barrier()

    pltpu.sync_copy(dst3d_hbm.at[wid], didx_v)
    _fire_drain(lambda j: ones_v, deg_sh, didx_v, sem, NCH)

    nrows = jnp.where(s < NS - 1, 24, 32)

    @pl.when(jnp.logical_and(c == 0, s < NS - 1))
    def _():
        pltpu.sync_copy(batch2d_hbm.at[pl.ds(s * 24, 24)],
                        didx_v.at[pl.ds(0, 24)])

    @pl.when(jnp.logical_and(c == 0, s == NS - 1))
    def _():
        pltpu.sync_copy(batch2d_hbm.at[pl.ds((NS - 1) * 24, 32)],
                        didx_v.at[pl.ds(0, 32)])

    @pl.when(c == 0)
    def _():
        _fire_drain(lambda j: ones_v, cnt_sh, didx_v, sem, nrows)

    plsc.subcore_barrier()

    pltpu.sync_copy(deg_sh.at[pl.ds(s * SL, SL)], zbuf_v)
    pltpu.sync_copy(zbuf_v, hist_hbm.at[pl.ds(c * NPAD + s * SL, SL)])

    @pl.when(jnp.logical_and(c == 0, s == 0))
    def _():
        pltpu.sync_copy(cnt_sh, zbuf_v.at[pl.ds(0, 256)])
        pltpu.sync_copy(zbuf_v.at[pl.ds(0, 256)], cnt_hbm)


@functools.partial(
    pl.kernel,
    out_type=jax.ShapeDtypeStruct((2 * NPAD,), jnp.float32),
    mesh=_mesh,
    scratch_types=[
        pltpu.VMEM((NCH, 128), jnp.int32),
        pltpu.VMEM((NCH, 128), jnp.int32),
        pltpu.VMEM((EPT,), jnp.float32),
        pltpu.VMEM_SHARED((NPAD,), jnp.float32),
        pltpu.VMEM_SHARED((NPAD,), jnp.float32),
        pltpu.SemaphoreType.DMA,
    ],
    compiler_params=_sc_params,
)
def _gs_kernel(src3d_hbm, dst3d_hbm, hist_hbm, x_hbm, t_hbm,
               sidx_v, didx_v, gvals_v, a_sh, t_sh, sem):
    c = lax.axis_index("c")
    s = lax.axis_index("s")
    wid = c * NS + s

    _fill(gvals_v, SL // 16, 0.0)
    pltpu.sync_copy(gvals_v.at[pl.ds(0, SL)], t_sh.at[pl.ds(s * SL, SL)])

    pltpu.sync_copy(hist_hbm.at[pl.ds(s * SL, SL)], gvals_v.at[pl.ds(0, SL)])
    pltpu.sync_copy(hist_hbm.at[pl.ds(NPAD + s * SL, SL)],
                    gvals_v.at[pl.ds(SL, SL)])
    pltpu.sync_copy(x_hbm.at[pl.ds(s * SL, SL)], gvals_v.at[pl.ds(2 * SL, SL)])

    def newton(k, _):
        d = (gvals_v[pl.ds(k * 16, 16)]
             + gvals_v[pl.ds(SL + k * 16, 16)] + 1.0)
        i = jnp.int32(0x5F3759DF) - lax.shift_right_logical(
            plsc.bitcast(d, jnp.int32), 1)
        y = plsc.bitcast(i, jnp.float32)
        y = y * (1.5 - 0.5 * d * y * y)
        y = y * (1.5 - 0.5 * d * y * y)
        y = y * (1.5 - 0.5 * d * y * y)
        gvals_v[pl.ds(3 * SL + k * 16, 16)] = (
            gvals_v[pl.ds(2 * SL + k * 16, 16)] * y)
        return 0
    lax.fori_loop(0, SL // 16, newton, 0)

    pltpu.sync_copy(gvals_v.at[pl.ds(3 * SL, SL)], a_sh.at[pl.ds(s * SL, SL)])
    plsc.subcore_barrier()

    pltpu.sync_copy(src3d_hbm.at[wid], sidx_v)
    pltpu.sync_copy(dst3d_hbm.at[wid], didx_v)

    def gfire(j, _):
        pltpu.async_copy(a_sh.at[sidx_v.at[j]],
                         gvals_v.at[pl.ds(j * 128, 128)], sem)
        return 0
    lax.fori_loop(0, NCH, gfire, 0)

    def gdrain(j, _):
        pltpu.make_async_copy(a_sh.at[sidx_v.at[0]],
                              gvals_v.at[pl.ds(0, 128)], sem).wait()
        return 0
    lax.fori_loop(0, NCH, gdrain, 0)

    _fire_drain(lambda j: gvals_v.at[pl.ds(j * 128, 128)],
                t_sh, didx_v, sem, NCH)
    plsc.subcore_barrier()

    pltpu.sync_copy(t_sh.at[pl.ds(s * SL, SL)], gvals_v.at[pl.ds(0, SL)])
    pltpu.sync_copy(gvals_v.at[pl.ds(0, SL)],
                    t_hbm.at[pl.ds(c * NPAD + s * SL, SL)])


def _head_body(t0_ref, t1_ref, h0_ref, h1_ref, x_ref, b_ref, cnt_ref,
               w1t_ref, b1t_ref, w2_ref, b2_ref, w3a_ref, w3b_ref, b3_ref,
               w4_ref, b4_ref, y_ref, out_ref, acc):
    i = pl.program_id(0)

    @pl.when(i == 0)
    def _():
        acc[...] = jnp.zeros_like(acc)

    d = h0_ref[0] + h1_ref[0] + 1.0
    t = t0_ref[0] + t1_ref[0]
    s = lax.rsqrt(d) * t + x_ref[0] / d
    b = b_ref[0].astype(jnp.bfloat16)

    mat_t = jnp.maximum(
        jnp.broadcast_to(w1t_ref[...], (G, 1024))
        * jnp.broadcast_to(s, (G, 1024))
        + jnp.broadcast_to(b1t_ref[...], (G, 1024)),
        0.0).astype(jnp.bfloat16)

    gid = lax.broadcasted_iota(jnp.int32, (G, 1), 0).astype(jnp.bfloat16)
    oh = (jnp.broadcast_to(gid, (G, 1024))
          == jnp.broadcast_to(b, (G, 1024))).astype(jnp.bfloat16)

    acc[...] += lax.dot_general(
        oh, mat_t, (((1,), (1,)), ((), ())),
        preferred_element_type=jnp.float32)

    @pl.when(i == TCG - 1)
    def _():
        pooled = acc[...] / jnp.maximum(cnt_ref[...], 1.0)
        emb = jnp.maximum(
            jnp.dot(pooled, w2_ref[...],
                    precision=lax.Precision.HIGHEST) + b2_ref[...], 0.0)
        h3 = jnp.maximum(
            jnp.dot(emb, w3a_ref[...], precision=lax.Precision.HIGHEST)
            + y_ref[...] * w3b_ref[...] + b3_ref[...], 0.0)
        out_ref[...] = (
            jnp.dot(h3, w4_ref[...], precision=lax.Precision.HIGHEST)
            + b4_ref[...])


def _head_call(t0, t1, h0, h1, x2d, b2d, cnt_col, w1t, b1t, w2, b2r,
               w3a, w3b, b3r, w4, b4r, ycol):
    blk = pl.BlockSpec((1, 1, 1024), lambda i: (i, 0, 0))
    full = lambda shape: pl.BlockSpec(shape, lambda i: tuple(0 for _ in shape))
    return pl.pallas_call(
        _head_body,
        grid=(TCG,),
        in_specs=[
            blk, blk, blk, blk, blk, blk,
            full((G, 1)),
            full((G, 1)), full((G, 1)),
            full((128, 64)), full((1, 64)),
            full((64, 32)), full((1, 32)), full((1, 32)),
            full((32, 1)), full((1, 1)),
            full((G, 1)),
        ],
        out_specs=pl.BlockSpec((G, 1), lambda i: (0, 0)),
        out_shape=jax.ShapeDtypeStruct((G, 1), jnp.float32),
        scratch_shapes=[pltpu.VMEM((G, G), jnp.float32)],
    )(t0, t1, h0, h1, x2d, b2d, cnt_col, w1t, b1t, w2, b2r, w3a, w3b, b3r,
      w4, b4r, ycol)


def kernel(x, edge_index, batch, y, W1, b1, W2, b2, W3, b3, W4, b4):
    pad_idx = jnp.full((EPAD - E,), NPAD - 1, dtype=jnp.int32)
    src3d = jnp.concatenate([edge_index[0], pad_idx]).reshape(NW, NCH, 128)
    dst3d = jnp.concatenate([edge_index[1], pad_idx]).reshape(NW, NCH, 128)

    x_flat = jnp.pad(x.reshape(-1), (0, NPAD - N))
    batch_p = jnp.pad(batch, (0, NPAD - N), constant_values=G)

    hist, cnt = _deg_kernel(dst3d, batch_p.reshape(ROWS, 128))
    t = _gs_kernel(src3d, dst3d, hist, x_flat)

    out = _head_call(
        t[:NPAD].reshape(TCG, 1, 1024),
        t[NPAD:].reshape(TCG, 1, 1024),
        hist[:NPAD].reshape(TCG, 1, 1024),
        hist[NPAD:].reshape(TCG, 1, 1024),
        x_flat.reshape(TCG, 1, 1024),
        batch_p.reshape(TCG, 1, 1024),
        cnt[:G].reshape(G, 1),
        W1.reshape(G, 1), b1.reshape(G, 1),
        W2, b2.reshape(1, 64),
        W3[:64], W3[64:65], b3.reshape(1, 32),
        W4, b4.reshape(1, 1),
        y.reshape(G, 1),
    )
    return out.reshape(-1)


# speedup vs baseline: 192.5297x; 1.3757x over previous
import functools

import jax
import jax.numpy as jnp
from jax import lax
from jax.experimental import pallas as pl
from jax.experimental.pallas import tpu as pltpu
from jax.experimental.pallas import tpu_sc as plsc

N = 50000
E = 800000
G = 128

NC = 2
NS = 16
NW = NC * NS

NPAD = 50176
ROWS = NPAD // 128
SL = NPAD // NS
ER = E // 128
NCH = 200
TAIL_CH = ER - (NW - 1) * NCH
TCG = ROWS // 8

_mesh = plsc.VectorSubcoreMesh(
    core_axis_name="c", subcore_axis_name="s", num_cores=NC, num_subcores=NS)
_sc_params = pltpu.CompilerParams(needs_layout_passes=False)


def _fill(buf, n16, value):
    def body(i, _):
        buf[pl.ds(i * 16, 16)] = jnp.full((16,), value, jnp.float32)
        return 0
    lax.fori_loop(0, n16, body, 0)


def _fire_drain(vals_at, acc_sh, didx_v, sem, nch):
    def fire(j, _):
        pltpu.async_copy(vals_at(j), acc_sh.at[didx_v.at[j]], sem, add=True)
        return 0
    lax.fori_loop(0, nch, fire, 0)

    def drain(j, _):
        pltpu.make_async_copy(vals_at(0), acc_sh.at[didx_v.at[0]], sem).wait()
        return 0
    lax.fori_loop(0, nch, drain, 0)


def _load_edge_rows(ei3_hbm, row, idx_v, wid):
    @pl.when(wid < NW - 1)
    def _():
        pltpu.sync_copy(ei3_hbm.at[row, pl.ds(wid * NCH, NCH)], idx_v)

    @pl.when(wid == NW - 1)
    def _():
        pltpu.sync_copy(ei3_hbm.at[row, pl.ds((NW - 1) * NCH, TAIL_CH)],
                        idx_v.at[pl.ds(0, TAIL_CH)])
    return jnp.where(wid == NW - 1, TAIL_CH, NCH)


@functools.partial(
    pl.kernel,
    out_type=(jax.ShapeDtypeStruct((2 * NPAD,), jnp.float32),
              jax.ShapeDtypeStruct((256,), jnp.float32)),
    mesh=_mesh,
    scratch_types=[
        pltpu.VMEM((NCH, 128), jnp.int32),
        pltpu.VMEM((128,), jnp.float32),
        pltpu.VMEM((SL,), jnp.float32),
        pltpu.VMEM_SHARED((NPAD,), jnp.float32),
        pltpu.VMEM_SHARED((256,), jnp.float32),
        pltpu.SemaphoreType.DMA,
    ],
    compiler_params=_sc_params,
)
def _deg_kernel(ei3_hbm, batch2d_hbm, hist_hbm, cnt_hbm,
                didx_v, ones_v, zbuf_v, deg_sh, cnt_sh, sem):
    c = lax.axis_index("c")
    s = lax.axis_index("s")
    wid = c * NS + s

    _fill(zbuf_v, SL // 16, 0.0)
    pltpu.sync_copy(zbuf_v, deg_sh.at[pl.ds(s * SL, SL)])

    @pl.when(jnp.logical_and(c == 0, s == 0))
    def _():
        pltpu.sync_copy(zbuf_v.at[pl.ds(0, 256)], cnt_sh)

    _fill(ones_v, 8, 1.0)
    plsc.subcore_barrier()

    nch = _load_edge_rows(ei3_hbm, 1, didx_v, wid)
    _fire_drain(lambda j: ones_v, deg_sh, didx_v, sem, nch)

    nrows = jnp.where(s < NS - 1, 24, 32)

    @pl.when(jnp.logical_and(c == 0, s < NS - 1))
    def _():
        pltpu.sync_copy(batch2d_hbm.at[pl.ds(s * 24, 24)],
                        didx_v.at[pl.ds(0, 24)])

    @pl.when(jnp.logical_and(c == 0, s == NS - 1))
    def _():
        pltpu.sync_copy(batch2d_hbm.at[pl.ds((NS - 1) * 24, 32)],
                        didx_v.at[pl.ds(0, 32)])

    @pl.when(c == 0)
    def _():
        _fire_drain(lambda j: ones_v, cnt_sh, didx_v, sem, nrows)

    plsc.subcore_barrier()

    pltpu.sync_copy(deg_sh.at[pl.ds(s * SL, SL)], zbuf_v)
    pltpu.sync_copy(zbuf_v, hist_hbm.at[pl.ds(c * NPAD + s * SL, SL)])

    @pl.when(jnp.logical_and(c == 0, s == 0))
    def _():
        pltpu.sync_copy(cnt_sh, zbuf_v.at[pl.ds(0, 256)])
        pltpu.sync_copy(zbuf_v.at[pl.ds(0, 256)], cnt_hbm)


@functools.partial(
    pl.kernel,
    out_type=jax.ShapeDtypeStruct((2 * NPAD,), jnp.float32),
    mesh=_mesh,
    scratch_types=[
        pltpu.VMEM((NCH, 128), jnp.int32),
        pltpu.VMEM((NCH, 128), jnp.int32),
        pltpu.VMEM((NCH * 128,), jnp.float32),
        pltpu.VMEM_SHARED((NPAD,), jnp.float32),
        pltpu.VMEM_SHARED((NPAD,), jnp.float32),
        pltpu.SemaphoreType.DMA,
    ],
    compiler_params=_sc_params,
)
def _gs_kernel(ei3_hbm, hist_hbm, x_hbm, t_hbm,
               sidx_v, didx_v, gvals_v, a_sh, t_sh, sem):
    c = lax.axis_index("c")
    s = lax.axis_index("s")
    wid = c * NS + s

    _fill(gvals_v, SL // 16, 0.0)
    pltpu.sync_copy(gvals_v.at[pl.ds(0, SL)], t_sh.at[pl.ds(s * SL, SL)])

    pltpu.sync_copy(hist_hbm.at[pl.ds(s * SL, SL)], gvals_v.at[pl.ds(0, SL)])
    pltpu.sync_copy(hist_hbm.at[pl.ds(NPAD + s * SL, SL)],
                    gvals_v.at[pl.ds(SL, SL)])
    pltpu.sync_copy(x_hbm.at[pl.ds(s * SL, SL)], gvals_v.at[pl.ds(2 * SL, SL)])

    def newton(k, _):
        d = (gvals_v[pl.ds(k * 16, 16)]
             + gvals_v[pl.ds(SL + k * 16, 16)] + 1.0)
        i = jnp.int32(0x5F3759DF) - lax.shift_right_logical(
            plsc.bitcast(d, jnp.int32), 1)
        y = plsc.bitcast(i, jnp.float32)
        y = y * (1.5 - 0.5 * d * y * y)
        y = y * (1.5 - 0.5 * d * y * y)
        y = y * (1.5 - 0.5 * d * y * y)
        gvals_v[pl.ds(3 * SL + k * 16, 16)] = (
            gvals_v[pl.ds(2 * SL + k * 16, 16)] * y)
        return 0
    lax.fori_loop(0, SL // 16, newton, 0)

    pltpu.sync_copy(gvals_v.at[pl.ds(3 * SL, SL)], a_sh.at[pl.ds(s * SL, SL)])
    plsc.subcore_barrier()

    _load_edge_rows(ei3_hbm, 0, sidx_v, wid)
    nch = _load_edge_rows(ei3_hbm, 1, didx_v, wid)

    def gfire(j, _):
        pltpu.async_copy(a_sh.at[sidx_v.at[j]],
                         gvals_v.at[pl.ds(j * 128, 128)], sem)
        return 0
    lax.fori_loop(0, nch, gfire, 0)

    def gdrain(j, _):
        pltpu.make_async_copy(a_sh.at[sidx_v.at[0]],
                              gvals_v.at[pl.ds(0, 128)], sem).wait()
        return 0
    lax.fori_loop(0, nch, gdrain, 0)

    _fire_drain(lambda j: gvals_v.at[pl.ds(j * 128, 128)],
                t_sh, didx_v, sem, nch)
    plsc.subcore_barrier()

    pltpu.sync_copy(t_sh.at[pl.ds(s * SL, SL)], gvals_v.at[pl.ds(0, SL)])
    pltpu.sync_copy(gvals_v.at[pl.ds(0, SL)],
                    t_hbm.at[pl.ds(c * NPAD + s * SL, SL)])


def _head_body(t_ref, h_ref, x_ref, b_ref, cnt_ref,
               w1t_ref, b1t_ref, w2_ref, b2_ref, w3a_ref, w3b_ref, b3_ref,
               w4_ref, b4_ref, y_ref, out_ref, acc):
    i = pl.program_id(0)

    @pl.when(i == 0)
    def _():
        acc[...] = jnp.zeros_like(acc)

    d = h_ref[0, 0] + h_ref[1, 0] + 1.0
    t = t_ref[0, 0] + t_ref[1, 0]
    s = lax.rsqrt(d) * t + x_ref[0] / d
    b = b_ref[0].astype(jnp.bfloat16)

    mat_t = jnp.maximum(
        jnp.broadcast_to(w1t_ref[...], (G, 1024))
        * jnp.broadcast_to(s, (G, 1024))
        + jnp.broadcast_to(b1t_ref[...], (G, 1024)),
        0.0).astype(jnp.bfloat16)

    gid = lax.broadcasted_iota(jnp.int32, (G, 1), 0).astype(jnp.bfloat16)
    oh = (jnp.broadcast_to(gid, (G, 1024))
          == jnp.broadcast_to(b, (G, 1024))).astype(jnp.bfloat16)

    acc[...] += lax.dot_general(
        oh, mat_t, (((1,), (1,)), ((), ())),
        preferred_element_type=jnp.float32)

    @pl.when(i == TCG - 1)
    def _():
        pooled = acc[...] / jnp.maximum(cnt_ref[...], 1.0)
        emb = jnp.maximum(
            jnp.dot(pooled, w2_ref[...],
                    precision=lax.Precision.HIGHEST) + b2_ref[...], 0.0)
        h3 = jnp.maximum(
            jnp.dot(emb, w3a_ref[...], precision=lax.Precision.HIGHEST)
            + y_ref[...] * w3b_ref[...] + b3_ref[...], 0.0)
        out_ref[...] = (
            jnp.dot(h3, w4_ref[...], precision=lax.Precision.HIGHEST)
            + b4_ref[...])


def _head_call(t4, h4, x3, b3d, cnt_col, w1t, b1t, w2, b2r,
               w3a, w3b, b3r, w4, b4r, ycol):
    blk = pl.BlockSpec((1, 1, 1024), lambda i: (i, 0, 0))
    blk2 = pl.BlockSpec((2, 1, 1, 1024), lambda i: (0, i, 0, 0))
    full = lambda shape: pl.BlockSpec(shape, lambda i: tuple(0 for _ in shape))
    return pl.pallas_call(
        _head_body,
        grid=(TCG,),
        in_specs=[
            blk2, blk2, blk, blk,
            full((G, 1)),
            full((G, 1)), full((G, 1)),
            full((128, 64)), full((1, 64)),
            full((64, 32)), full((1, 32)), full((1, 32)),
            full((32, 1)), full((1, 1)),
            full((G, 1)),
        ],
        out_specs=pl.BlockSpec((G, 1), lambda i: (0, 0)),
        out_shape=jax.ShapeDtypeStruct((G, 1), jnp.float32),
        scratch_shapes=[pltpu.VMEM((G, G), jnp.float32)],
    )(t4, h4, x3, b3d, cnt_col, w1t, b1t, w2, b2r, w3a, w3b, b3r,
      w4, b4r, ycol)


def kernel(x, edge_index, batch, y, W1, b1, W2, b2, W3, b3, W4, b4):
    ei3 = edge_index.reshape(2, ER, 128)

    x_flat = jnp.pad(x.reshape(-1), (0, NPAD - N))
    batch_p = jnp.pad(batch, (0, NPAD - N), constant_values=G)

    hist, cnt = _deg_kernel(ei3, batch_p.reshape(ROWS, 128))
    t = _gs_kernel(ei3, hist, x_flat)

    out = _head_call(
        t.reshape(2, TCG, 1, 1024),
        hist.reshape(2, TCG, 1, 1024),
        x_flat.reshape(TCG, 1, 1024),
        batch_p.reshape(TCG, 1, 1024),
        cnt[:G].reshape(G, 1),
        W1.reshape(G, 1), b1.reshape(G, 1),
        W2, b2.reshape(1, 64),
        W3[:64], W3[64:65], b3.reshape(1, 32),
        W4, b4.reshape(1, 1),
        y.reshape(G, 1),
    )
    return out.reshape(-1)


# speedup vs baseline: 237.9783x; 1.2361x over previous
import functools

import jax
import jax.numpy as jnp
from jax import lax
from jax.experimental import pallas as pl
from jax.experimental.pallas import tpu as pltpu
from jax.experimental.pallas import tpu_sc as plsc

N = 50000
E = 800000
G = 128

NC = 2
NS = 16
NW = NC * NS

NPAD = 50176
ROWS = NPAD // 128
SL = NPAD // NS
ER = E // 128
NCH = 200
TAIL_CH = ER - (NW - 1) * NCH
TCG = 7
LW = NPAD // TCG

_mesh = plsc.VectorSubcoreMesh(
    core_axis_name="c", subcore_axis_name="s", num_cores=NC, num_subcores=NS)
_sc_params = pltpu.CompilerParams(needs_layout_passes=False,
                                  use_tc_tiling_on_sc=False)


def _fill(buf, n16, value):
    def body(i, _):
        buf[pl.ds(i * 16, 16)] = jnp.full((16,), value, jnp.float32)
        return 0
    lax.fori_loop(0, n16, body, 0)


def _fire_drain(vals_at, acc_sh, didx_v, sem, nch):
    def fire(j, _):
        pltpu.async_copy(vals_at(j), acc_sh.at[didx_v.at[j]], sem, add=True)
        return 0
    lax.fori_loop(0, nch, fire, 0)

    def drain(j, _):
        pltpu.make_async_copy(vals_at(0), acc_sh.at[didx_v.at[0]], sem).wait()
        return 0
    lax.fori_loop(0, nch, drain, 0)


def _load_edge_rows(ei3_hbm, row, idx_v, wid):
    @pl.when(wid < NW - 1)
    def _():
        pltpu.sync_copy(ei3_hbm.at[row, pl.ds(wid * NCH, NCH)], idx_v)

    @pl.when(wid == NW - 1)
    def _():
        pltpu.sync_copy(ei3_hbm.at[row, pl.ds((NW - 1) * NCH, TAIL_CH)],
                        idx_v.at[pl.ds(0, TAIL_CH)])
    return jnp.where(wid == NW - 1, TAIL_CH, NCH)


@functools.partial(
    pl.kernel,
    out_type=(jax.ShapeDtypeStruct((2 * NPAD,), jnp.float32),
              jax.ShapeDtypeStruct((256,), jnp.float32)),
    mesh=_mesh,
    scratch_types=[
        pltpu.VMEM((NCH, 128), jnp.int32),
        pltpu.VMEM((128,), jnp.float32),
        pltpu.VMEM((SL,), jnp.float32),
        pltpu.VMEM_SHARED((NPAD,), jnp.float32),
        pltpu.VMEM_SHARED((256,), jnp.float32),
        pltpu.SemaphoreType.DMA,
    ],
    compiler_params=_sc_params,
)
def _deg_kernel(ei3_hbm, batch2d_hbm, hist_hbm, cnt_hbm,
                didx_v, ones_v, zbuf_v, deg_sh, cnt_sh, sem):
    c = lax.axis_index("c")
    s = lax.axis_index("s")
    wid = c * NS + s

    _fill(zbuf_v, SL // 16, 0.0)
    pltpu.sync_copy(zbuf_v, deg_sh.at[pl.ds(s * SL, SL)])

    @pl.when(jnp.logical_and(c == 0, s == 0))
    def _():
        pltpu.sync_copy(zbuf_v.at[pl.ds(0, 256)], cnt_sh)

    _fill(ones_v, 8, 1.0)
    plsc.subcore_barrier()

    nch = _load_edge_rows(ei3_hbm, 1, didx_v, wid)
    _fire_drain(lambda j: ones_v, deg_sh, didx_v, sem, nch)

    nrows = jnp.where(s < NS - 1, 24, 32)

    @pl.when(jnp.logical_and(c == 0, s < NS - 1))
    def _():
        pltpu.sync_copy(batch2d_hbm.at[pl.ds(s * 24, 24)],
                        didx_v.at[pl.ds(0, 24)])

    @pl.when(jnp.logical_and(c == 0, s == NS - 1))
    def _():
        pltpu.sync_copy(batch2d_hbm.at[pl.ds((NS - 1) * 24, 32)],
                        didx_v.at[pl.ds(0, 32)])

    @pl.when(c == 0)
    def _():
        _fire_drain(lambda j: ones_v, cnt_sh, didx_v, sem, nrows)

    plsc.subcore_barrier()

    pltpu.sync_copy(deg_sh.at[pl.ds(s * SL, SL)], zbuf_v)
    pltpu.sync_copy(zbuf_v, hist_hbm.at[pl.ds(c * NPAD + s * SL, SL)])

    @pl.when(jnp.logical_and(c == 0, s == 0))
    def _():
        pltpu.sync_copy(cnt_sh, zbuf_v.at[pl.ds(0, 256)])
        pltpu.sync_copy(zbuf_v.at[pl.ds(0, 256)], cnt_hbm)


@functools.partial(
    pl.kernel,
    out_type=jax.ShapeDtypeStruct((2 * NPAD,), jnp.float32),
    mesh=_mesh,
    scratch_types=[
        pltpu.VMEM((NCH, 128), jnp.int32),
        pltpu.VMEM((NCH, 128), jnp.int32),
        pltpu.VMEM((NCH * 128,), jnp.float32),
        pltpu.VMEM_SHARED((NPAD,), jnp.float32),
        pltpu.VMEM_SHARED((NPAD,), jnp.float32),
        pltpu.SemaphoreType.DMA,
    ],
    compiler_params=_sc_params,
)
def _gs_kernel(ei3_hbm, hist_hbm, x_hbm, t_hbm,
               sidx_v, didx_v, gvals_v, a_sh, t_sh, sem):
    c = lax.axis_index("c")
    s = lax.axis_index("s")
    wid = c * NS + s

    _fill(gvals_v, SL // 16, 0.0)
    pltpu.sync_copy(gvals_v.at[pl.ds(0, SL)], t_sh.at[pl.ds(s * SL, SL)])

    pltpu.sync_copy(hist_hbm.at[pl.ds(s * SL, SL)], gvals_v.at[pl.ds(0, SL)])
    pltpu.sync_copy(hist_hbm.at[pl.ds(NPAD + s * SL, SL)],
                    gvals_v.at[pl.ds(SL, SL)])
    pltpu.sync_copy(x_hbm.at[pl.ds(s * SL, SL)], gvals_v.at[pl.ds(2 * SL, SL)])

    def newton(k, _):
        d = (gvals_v[pl.ds(k * 16, 16)]
             + gvals_v[pl.ds(SL + k * 16, 16)] + 1.0)
        i = jnp.int32(0x5F3759DF) - lax.shift_right_logical(
            plsc.bitcast(d, jnp.int32), 1)
        y = plsc.bitcast(i, jnp.float32)
        y = y * (1.5 - 0.5 * d * y * y)
        y = y * (1.5 - 0.5 * d * y * y)
        y = y * (1.5 - 0.5 * d * y * y)
        gvals_v[pl.ds(3 * SL + k * 16, 16)] = (
            gvals_v[pl.ds(2 * SL + k * 16, 16)] * y)
        return 0
    lax.fori_loop(0, SL // 16, newton, 0)

    pltpu.sync_copy(gvals_v.at[pl.ds(3 * SL, SL)], a_sh.at[pl.ds(s * SL, SL)])
    plsc.subcore_barrier()

    _load_edge_rows(ei3_hbm, 0, sidx_v, wid)
    nch = _load_edge_rows(ei3_hbm, 1, didx_v, wid)

    def gfire(j, _):
        pltpu.async_copy(a_sh.at[sidx_v.at[j]],
                         gvals_v.at[pl.ds(j * 128, 128)], sem)
        return 0
    lax.fori_loop(0, nch, gfire, 0)

    def gdrain(j, _):
        pltpu.make_async_copy(a_sh.at[sidx_v.at[0]],
                              gvals_v.at[pl.ds(0, 128)], sem).wait()
        return 0
    lax.fori_loop(0, nch, gdrain, 0)

    _fire_drain(lambda j: gvals_v.at[pl.ds(j * 128, 128)],
                t_sh, didx_v, sem, nch)
    plsc.subcore_barrier()

    pltpu.sync_copy(t_sh.at[pl.ds(s * SL, SL)], gvals_v.at[pl.ds(0, SL)])
    pltpu.sync_copy(gvals_v.at[pl.ds(0, SL)],
                    t_hbm.at[pl.ds(c * NPAD + s * SL, SL)])


def _head_body(t_ref, h_ref, x_ref, b_ref, cnt_ref,
               w1t_ref, b1t_ref, w2_ref, b2_ref, w3a_ref, w3b_ref, b3_ref,
               w4_ref, b4_ref, y_ref, out_ref, acc):
    i = pl.program_id(0)

    @pl.when(i == 0)
    def _():
        acc[...] = jnp.zeros_like(acc)

    d = h_ref[0, 0] + h_ref[1, 0] + 1.0
    t = t_ref[0, 0] + t_ref[1, 0]
    s = lax.rsqrt(d) * t + x_ref[0] / d
    b = b_ref[0].astype(jnp.bfloat16)

    mat_t = jnp.maximum(
        jnp.broadcast_to(w1t_ref[...], (G, LW))
        * jnp.broadcast_to(s, (G, LW))
        + jnp.broadcast_to(b1t_ref[...], (G, LW)),
        0.0).astype(jnp.bfloat16)

    gid = lax.broadcasted_iota(jnp.int32, (G, 1), 0).astype(jnp.bfloat16)
    oh = (jnp.broadcast_to(gid, (G, LW))
          == jnp.broadcast_to(b, (G, LW))).astype(jnp.bfloat16)

    acc[...] += lax.dot_general(
        oh, mat_t, (((1,), (1,)), ((), ())),
        preferred_element_type=jnp.float32)

    @pl.when(i == TCG - 1)
    def _():
        pooled = acc[...] / jnp.maximum(cnt_ref[...], 1.0)
        emb = jnp.maximum(
            jnp.dot(pooled, w2_ref[...],
                    precision=lax.Precision.HIGHEST) + b2_ref[...], 0.0)
        h3 = jnp.maximum(
            jnp.dot(emb, w3a_ref[...], precision=lax.Precision.HIGHEST)
            + y_ref[...] * w3b_ref[...] + b3_ref[...], 0.0)
        out_ref[...] = (
            jnp.dot(h3, w4_ref[...], precision=lax.Precision.HIGHEST)
            + b4_ref[...])


def _head_call(t4, h4, x3, b3d, cnt_col, w1t, b1t, w2, b2r,
               w3a, w3b, b3r, w4, b4r, ycol):
    blk = pl.BlockSpec((1, 1, LW), lambda i: (i, 0, 0))
    blk2 = pl.BlockSpec((2, 1, 1, LW), lambda i: (0, i, 0, 0))
    full = lambda shape: pl.BlockSpec(shape, lambda i: tuple(0 for _ in shape))
    return pl.pallas_call(
        _head_body,
        grid=(TCG,),
        in_specs=[
            blk2, blk2, blk, blk,
            full((G, 1)),
            full((G, 1)), full((G, 1)),
            full((128, 64)), full((1, 64)),
            full((64, 32)), full((1, 32)), full((1, 32)),
            full((32, 1)), full((1, 1)),
            full((G, 1)),
        ],
        out_specs=pl.BlockSpec((G, 1), lambda i: (0, 0)),
        out_shape=jax.ShapeDtypeStruct((G, 1), jnp.float32),
        scratch_shapes=[pltpu.VMEM((G, G), jnp.float32)],
    )(t4, h4, x3, b3d, cnt_col, w1t, b1t, w2, b2r, w3a, w3b, b3r,
      w4, b4r, ycol)


def kernel(x, edge_index, batch, y, W1, b1, W2, b2, W3, b3, W4, b4):
    ei3 = edge_index.reshape(2, ER, 128)

    x_flat = jnp.pad(x.reshape(-1), (0, NPAD - N))
    batch_p = jnp.pad(batch, (0, NPAD - N), constant_values=G)

    hist, cnt = _deg_kernel(ei3, batch_p.reshape(ROWS, 128))
    t = _gs_kernel(ei3, hist, x_flat)

    out = _head_call(
        t.reshape(2, TCG, 1, LW),
        hist.reshape(2, TCG, 1, LW),
        x_flat.reshape(TCG, 1, LW),
        batch_p.reshape(TCG, 1, LW),
        cnt[:G].reshape(G, 1),
        W1.reshape(G, 1), b1.reshape(G, 1),
        W2, b2.reshape(1, 64),
        W3[:64], W3[64:65], b3.reshape(1, 32),
        W4, b4.reshape(1, 1),
        y.reshape(G, 1),
    )
    return out.reshape(-1)
